# baseline (speedup 1.0000x reference)
import functools

import jax
import jax.numpy as jnp
from jax.experimental import pallas as pl
from jax.experimental.pallas import tpu as pltpu

OBS_DIM = 55
A1_DIM = 10
A2_DIM = 10
IN_DIM = OBS_DIM + A1_DIM + A2_DIM
HIDDEN = 128
OUT_DIM = 35
NEG_SLOPE = 0.01

_MAX_TILE_B = 2048
_SINGLE_STEP_MAX_B = 511


def _mlp_kernel(obs_ref, a1_ref, a2_ref,
                w1o_ref, w1a1_ref, w1a2_ref, b1_ref,
                w2_ref, b2_ref,
                w3_ref, b3_ref,
                o_ref):
    f32 = jnp.float32
    h = (jnp.dot(obs_ref[...].astype(jnp.bfloat16), w1o_ref[...],
                 preferred_element_type=f32)
         + jnp.dot(a1_ref[...].astype(jnp.bfloat16), w1a1_ref[...],
                   preferred_element_type=f32)
         + jnp.dot(a2_ref[...].astype(jnp.bfloat16), w1a2_ref[...],
                   preferred_element_type=f32)
         + b1_ref[...])

    h = jnp.dot(h.astype(jnp.bfloat16), w2_ref[...],
                preferred_element_type=f32) + b2_ref[...]
    h = jnp.where(h >= 0, h, NEG_SLOPE * h)

    o_ref[...] = (jnp.dot(h.astype(jnp.bfloat16), w3_ref[...],
                          preferred_element_type=f32)
                  + b3_ref[...]).astype(o_ref.dtype)


def _choose_tiling(B):
    if B <= _SINGLE_STEP_MAX_B:
        return 1, B
    n_steps = max(2, pl.cdiv(B, _MAX_TILE_B))
    tile_b = pl.cdiv(B, n_steps)
    tile_b = ((tile_b + 7) // 8) * 8
    return n_steps, tile_b


@functools.partial(jax.jit, static_argnames=())
def kernel(observation, action_j1, action_j2, w1o, w1a, b1, w2, b2, w3, b3):
    B = observation.shape[0]

    bf16 = jnp.bfloat16
    w1o_c = w1o.astype(bf16)
    w1a1_c = w1a[:A1_DIM, :].astype(bf16)
    w1a2_c = w1a[A1_DIM:, :].astype(bf16)
    w2_c = w2.astype(bf16)
    w3_c = w3.astype(bf16)
    b1_c = b1.astype(jnp.float32)
    b2_c = b2.astype(jnp.float32)
    b3_c = b3.astype(jnp.float32)

    n_steps, tile_b = _choose_tiling(B)
    Bp = n_steps * tile_b
    pad = Bp - B
    if pad:
        observation = jnp.pad(observation, ((0, pad), (0, 0)))
        action_j1 = jnp.pad(action_j1, ((0, pad), (0, 0)))
        action_j2 = jnp.pad(action_j2, ((0, pad), (0, 0)))

    def batch_spec(feat):
        return pl.BlockSpec((tile_b, feat), lambda i: (i, 0))

    def resident_spec(arr):
        return pl.BlockSpec(arr.shape, lambda i: (0, 0))

    weight_bytes = (2 * (w1o_c.size + w1a1_c.size + w1a2_c.size
                         + w2_c.size + w3_c.size)
                    + 4 * (b1_c.size + b2_c.size + b3_c.size))
    cost = pl.CostEstimate(
        flops=2 * Bp * (IN_DIM * HIDDEN + HIDDEN * HIDDEN + HIDDEN * OUT_DIM),
        transcendentals=0,
        bytes_accessed=Bp * 4 * (IN_DIM + OUT_DIM) + weight_bytes)

    out = pl.pallas_call(
        _mlp_kernel,
        out_shape=jax.ShapeDtypeStruct((Bp, OUT_DIM), jnp.float32),
        grid=(n_steps,),
        in_specs=[
            batch_spec(OBS_DIM), batch_spec(A1_DIM), batch_spec(A2_DIM),
            resident_spec(w1o_c), resident_spec(w1a1_c),
            resident_spec(w1a2_c), resident_spec(b1_c),
            resident_spec(w2_c), resident_spec(b2_c),
            resident_spec(w3_c), resident_spec(b3_c),
        ],
        out_specs=batch_spec(OUT_DIM),
        compiler_params=pltpu.CompilerParams(
            dimension_semantics=("parallel",)),
        cost_estimate=cost,
    )(observation, action_j1, action_j2,
      w1o_c, w1a1_c, w1a2_c, b1_c, w2_c, b2_c, w3_c, b3_c)

    return out[:B] if pad else out
